# baseline (speedup 1.0000x reference)
import functools

import jax
import jax.numpy as jnp
from jax import lax
from jax.experimental import pallas as pl
from jax.experimental.pallas import tpu as pltpu
from jax.experimental.pallas import tpu_sc as plsc

B, N_TOK, D = 8, 24576, 128
GRID = 49152
NC, NS, L = 2, 16, 16
NW = NC * NS
QPB = NW // B
RANGE = GRID // QPB
CHUNK = 128
NBLK = RANGE // CHUNK
WIN = 2048
NWIN = N_TOK // WIN
VPW = WIN // L
SENT = 0x7FFFFFFF

MROWS = 2048
MBLK = GRID // MROWS
MGRP = 8

_sc_mesh = lambda: plsc.VectorSubcoreMesh(
    core_axis_name="c", subcore_axis_name="s",
    num_cores=NC, num_subcores=NS)


def _winner_lists(idx):
  @functools.partial(
      pl.kernel,
      out_type=(
          jax.ShapeDtypeStruct((NW, RANGE), jnp.int32),
          jax.ShapeDtypeStruct((NW, NBLK, CHUNK), jnp.int32),
          jax.ShapeDtypeStruct((NW, L), jnp.int32),
      ),
      mesh=_sc_mesh(),
      compiler_params=pltpu.CompilerParams(needs_layout_passes=False),
      cost_estimate=pl.CostEstimate(
          flops=40_000_000, bytes_accessed=8_000_000, transcendentals=0),
      scratch_types=[
          pltpu.VMEM((2, WIN), jnp.int32),
          pltpu.VMEM((RANGE,), jnp.int32),
          pltpu.VMEM((L,), jnp.int32),
          pltpu.VMEM((RANGE + L,), jnp.int32),
          pltpu.VMEM((RANGE + L,), jnp.int32),
          pltpu.VMEM((NBLK, CHUNK), jnp.int32),
          pltpu.VMEM((L,), jnp.int32),
          pltpu.SemaphoreType.DMA,
      ],
  )
  def body(idx_hbm, wtok_hbm, wcell_hbm, cnt_hbm, idx_win, inv, bounce,
           wtok, wcell, wcell2, cnt_v, sem_i):
    wid = lax.axis_index("s") * NC + lax.axis_index("c")
    b = wid // QPB
    base = (wid % QPB) * RANGE

    iota = lax.iota(jnp.int32, L)
    neg16 = jnp.full((L,), -1, jnp.int32)
    shift_idx = jnp.minimum(iota + 1, L - 1)

    def init_inv(j, _):
      inv[pl.ds(j * L, L)] = neg16
      return 0
    lax.fori_loop(0, RANGE // L, init_inv, 0)

    pltpu.async_copy(idx_hbm.at[b, pl.ds(0, WIN)], idx_win.at[0], sem_i)
    for w in range(NWIN):
      if w + 1 < NWIN:
        pltpu.async_copy(idx_hbm.at[b, pl.ds((w + 1) * WIN, WIN)],
                         idx_win.at[(w + 1) % 2], sem_i)
      pltpu.make_async_copy(idx_hbm.at[b, pl.ds(w * WIN, WIN)],
                            idx_win.at[w % 2], sem_i).wait()

      def vreg_body(k, _, w=w):
        v = idx_win[w % 2, pl.ds(k * L, L)]
        local = v - base
        m = (local >= 0) & (local < RANGE)
        p = (w * WIN + k * L) + iota
        key = jnp.where(m, (local << 15) | p, SENT)
        skey, _ = plsc.sort_key_val(key, key)
        bounce[...] = skey
        snext = plsc.load_gather(bounce, [shift_idx])
        keep = ((skey >> 15) != (snext >> 15)) | (iota == L - 1)
        valid = skey != SENT
        plsc.store_scatter(inv, [skey >> 15], skey & 0x7FFF,
                           mask=keep & valid)
        return 0
      lax.fori_loop(0, VPW, vreg_body, 0)

    def extract(j, cnt):
      v = inv[pl.ds(j * L, L)]
      m = v >= 0
      plsc.store_compressed(wtok.at[pl.ds(cnt, L)], v, mask=m)
      plsc.store_compressed(wcell.at[pl.ds(cnt, L)],
                            base + j * L + iota, mask=m)
      return cnt + jnp.max(plsc.all_reduce_population_count(m))
    cnt = lax.fori_loop(0, RANGE // L, extract, jnp.int32(0))

    @pl.when(cnt > 0)
    def _():
      nch = (cnt + CHUNK - 1) // CHUNK
      pend = nch * CHUNK
      ftok = plsc.load_gather(wtok, [jnp.zeros((L,), jnp.int32)])
      fcell = plsc.load_gather(wcell, [jnp.zeros((L,), jnp.int32)])
      start = (cnt // L) * L

      def pad(t, _):
        off = start + t * L

        @pl.when(off < pend)
        def _():
          m = (off + iota) >= cnt
          wtok[pl.ds(off, L)] = jnp.where(m, ftok, wtok[pl.ds(off, L)])
          wcell[pl.ds(off, L)] = jnp.where(m, fcell, wcell[pl.ds(off, L)])
        return 0
      lax.fori_loop(0, CHUNK // L, pad, 0)

      def repack(i, _):
        r = i // (CHUNK // L)
        c = (i % (CHUNK // L)) * L
        wcell2[r, pl.ds(c, L)] = wcell[pl.ds(i * L, L)]
        return 0
      lax.fori_loop(0, nch * (CHUNK // L), repack, 0)

    cnt_v[...] = jnp.full((L,), cnt, jnp.int32)

    pltpu.async_copy(wtok.at[pl.ds(0, RANGE)], wtok_hbm.at[wid], sem_i)
    pltpu.async_copy(wcell2, wcell_hbm.at[wid], sem_i)
    pltpu.async_copy(cnt_v, cnt_hbm.at[wid], sem_i)
    pltpu.make_async_copy(wtok.at[pl.ds(0, RANGE)], wtok_hbm.at[wid],
                          sem_i).wait()
    pltpu.make_async_copy(wcell2, wcell_hbm.at[wid], sem_i).wait()
    pltpu.make_async_copy(cnt_v, cnt_hbm.at[wid], sem_i).wait()

  return body(idx)


def _tc_memset():
  def body(o_ref):
    o_ref[...] = jnp.zeros_like(o_ref)

  return pl.pallas_call(
      body,
      out_shape=jax.ShapeDtypeStruct((B, GRID, D), jnp.float32),
      grid=(B, MBLK),
      out_specs=pl.BlockSpec((1, MROWS, D), lambda i, j: (i, j, 0)),
  )()


def _winner_move(tokens, wtok_all, wcell_all, cnt_all, out_ref):
  @functools.partial(
      pl.kernel,
      mesh=_sc_mesh(),
      compiler_params=pltpu.CompilerParams(needs_layout_passes=False),
      scratch_types=[
          pltpu.VMEM((RANGE,), jnp.int32),
          pltpu.VMEM((NBLK, CHUNK), jnp.int32),
          pltpu.VMEM((L,), jnp.int32),
          pltpu.VMEM((2, CHUNK, D), jnp.float32),
          pltpu.SemaphoreType.DMA,
          pltpu.SemaphoreType.DMA,
          pltpu.SemaphoreType.DMA,
      ],
  )
  def body(tokens_hbm, wtok_hbm, wcell_hbm, cnt_hbm, out_hbm,
           wtok, wcell2, cnt_v, rows, sem_t, sem_g, sem_s):
    wid = lax.axis_index("s") * NC + lax.axis_index("c")
    b = wid // QPB

    pltpu.async_copy(cnt_hbm.at[wid], cnt_v, sem_t)
    pltpu.async_copy(wtok_hbm.at[wid], wtok, sem_t)
    pltpu.async_copy(wcell_hbm.at[wid], wcell2, sem_t)
    pltpu.make_async_copy(cnt_hbm.at[wid], cnt_v, sem_t).wait()
    pltpu.make_async_copy(wtok_hbm.at[wid], wtok, sem_t).wait()
    pltpu.make_async_copy(wcell_hbm.at[wid], wcell2, sem_t).wait()

    cnt = jnp.max(plsc.load_gather(cnt_v, [jnp.zeros((L,), jnp.int32)]))

    out_b = out_hbm.at[b]
    tok_b = tokens_hbm.at[b]

    @pl.when(cnt > 0)
    def _():
      nch = (cnt + CHUNK - 1) // CHUNK

      def g_copy(ci, buf):
        return pltpu.make_async_copy(
            tok_b.at[wtok.at[pl.ds(ci * CHUNK, CHUNK)]], rows.at[buf], sem_g)

      def s_copy(ci, buf):
        return pltpu.make_async_copy(
            rows.at[buf], out_b.at[wcell2.at[ci]], sem_s)

      g_copy(0, 0).start()

      def move(ci, _):
        @pl.when(ci > 0)
        def _():
          s_copy(ci - 1, (ci - 1) % 2).wait()

        @pl.when(ci + 1 < nch)
        def _():
          g_copy(ci + 1, (ci + 1) % 2).start()
        g_copy(ci, ci % 2).wait()
        s_copy(ci, ci % 2).start()
        return 0
      lax.fori_loop(0, nch, move, 0)
      s_copy(nch - 1, (nch - 1) % 2).wait()

  body(tokens, wtok_all, wcell_all, cnt_all, out_ref)


def kernel(tokens, idx, grid_size):
  del grid_size
  idx32 = idx.astype(jnp.int32)
  wtok_all, wcell_all, cnt_all = _winner_lists(idx32)
  out_ref = jax.new_ref(jnp.zeros((B, GRID, D), jnp.float32))
  _winner_move(tokens, wtok_all, wcell_all, cnt_all, out_ref)
  return out_ref[...]


# speedup vs baseline: 1.2839x; 1.2839x over previous
import functools

import jax
import jax.numpy as jnp
from jax import lax
from jax.experimental import pallas as pl
from jax.experimental.pallas import tpu as pltpu
from jax.experimental.pallas import tpu_sc as plsc

B, N_TOK, D = 8, 24576, 128
GRID = 49152
NC, NS, L = 2, 16, 16
NW = NC * NS
QPB = NW // B
RANGE = GRID // QPB
CHUNK = 128
NBLK = RANGE // CHUNK
ZGRP = 8
WIN = 2048
NWIN = N_TOK // WIN
VPW = WIN // L
SENT = 0x7FFFFFFF


def _winner_scatter(tokens, idx):
  mesh = plsc.VectorSubcoreMesh(
      core_axis_name="c", subcore_axis_name="s",
      num_cores=NC, num_subcores=NS)

  @functools.partial(
      pl.kernel,
      out_type=jax.ShapeDtypeStruct((B, GRID, D), jnp.float32),
      mesh=mesh,
      compiler_params=pltpu.CompilerParams(needs_layout_passes=False),
      scratch_types=[
          pltpu.VMEM((2, WIN), jnp.int32),
          pltpu.VMEM((RANGE,), jnp.int32),
          pltpu.VMEM((L,), jnp.int32),
          pltpu.VMEM((RANGE + L,), jnp.int32),
          pltpu.VMEM((RANGE + L,), jnp.int32),
          pltpu.VMEM((NBLK, CHUNK), jnp.int32),
          pltpu.VMEM((4, CHUNK, D), jnp.float32),
          pltpu.SemaphoreType.DMA,
          pltpu.SemaphoreType.DMA,
          pltpu.SemaphoreType.DMA,
          pltpu.SemaphoreType.DMA,
      ],
  )
  def body(tokens_hbm, idx_hbm, out_hbm, idx_win, inv, bounce,
           wtok, wcell, wcell2, rows, sem_i, sem_z, sem_g, sem_s):
    wid = lax.axis_index("s") * NC + lax.axis_index("c")
    b = wid // QPB
    base = (wid % QPB) * RANGE

    iota = lax.iota(jnp.int32, L)
    zeros16f = jnp.zeros((L,), jnp.float32)
    neg16 = jnp.full((L,), -1, jnp.int32)
    shift_idx = jnp.minimum(iota + 1, L - 1)

    def init_inv(j, _):
      inv[pl.ds(j * L, L)] = neg16
      return 0
    lax.fori_loop(0, RANGE // L, init_inv, 0)

    def init_z(i, _):
      r = i // (D // L)
      c = (i % (D // L)) * L
      rows[0, r, pl.ds(c, L)] = zeros16f
      return 0
    lax.fori_loop(0, CHUNK * (D // L), init_z, 0)

    out_b = out_hbm.at[b]
    tok_b = tokens_hbm.at[b]

    def zero_start(g):
      for t in range(ZGRP):
        blk = g * ZGRP + t
        pltpu.async_copy(
            rows.at[0], out_b.at[pl.ds(base + blk * CHUNK, CHUNK)], sem_z)

    def zero_drain(g):
      for t in range(ZGRP):
        blk = g * ZGRP + t
        pltpu.make_async_copy(
            rows.at[0], out_b.at[pl.ds(base + blk * CHUNK, CHUNK)], sem_z).wait()

    pltpu.async_copy(idx_hbm.at[b, pl.ds(0, WIN)], idx_win.at[0], sem_i)
    for w in range(NWIN):
      if w + 1 < NWIN:
        pltpu.async_copy(idx_hbm.at[b, pl.ds((w + 1) * WIN, WIN)],
                         idx_win.at[(w + 1) % 2], sem_i)
      pltpu.make_async_copy(idx_hbm.at[b, pl.ds(w * WIN, WIN)],
                            idx_win.at[w % 2], sem_i).wait()
      zero_start(w)

      def vreg_body(k, _, w=w):
        v = idx_win[w % 2, pl.ds(k * L, L)]
        local = v - base
        m = (local >= 0) & (local < RANGE)
        p = (w * WIN + k * L) + iota
        key = jnp.where(m, (local << 15) | p, SENT)
        skey, _ = plsc.sort_key_val(key, key)
        bounce[...] = skey
        snext = plsc.load_gather(bounce, [shift_idx])
        keep = ((skey >> 15) != (snext >> 15)) | (iota == L - 1)
        valid = skey != SENT
        plsc.store_scatter(inv, [skey >> 15], skey & 0x7FFF,
                           mask=keep & valid)
        return 0
      lax.fori_loop(0, VPW, vreg_body, 0)
      if w >= 1:
        zero_drain(w - 1)

    def extract(j, cnt):
      v = inv[pl.ds(j * L, L)]
      m = v >= 0
      plsc.store_compressed(wtok.at[pl.ds(cnt, L)], v, mask=m)
      plsc.store_compressed(wcell.at[pl.ds(cnt, L)],
                            base + j * L + iota, mask=m)
      return cnt + jnp.max(plsc.all_reduce_population_count(m))
    cnt = lax.fori_loop(0, RANGE // L, extract, jnp.int32(0))
    zero_drain(NWIN - 1)

    @pl.when(cnt > 0)
    def _():
      nch = (cnt + CHUNK - 1) // CHUNK
      pend = nch * CHUNK
      ftok = plsc.load_gather(wtok, [jnp.zeros((L,), jnp.int32)])
      fcell = plsc.load_gather(wcell, [jnp.zeros((L,), jnp.int32)])
      start = (cnt // L) * L

      def pad(t, _):
        off = start + t * L

        @pl.when(off < pend)
        def _():
          m = (off + iota) >= cnt
          wtok[pl.ds(off, L)] = jnp.where(m, ftok, wtok[pl.ds(off, L)])
          wcell[pl.ds(off, L)] = jnp.where(m, fcell, wcell[pl.ds(off, L)])
        return 0
      lax.fori_loop(0, CHUNK // L, pad, 0)

      def repack(i, _):
        r = i // (CHUNK // L)
        c = (i % (CHUNK // L)) * L
        wcell2[r, pl.ds(c, L)] = wcell[pl.ds(i * L, L)]
        return 0
      lax.fori_loop(0, nch * (CHUNK // L), repack, 0)

      def g_copy(ci, buf):
        return pltpu.make_async_copy(
            tok_b.at[wtok.at[pl.ds(ci * CHUNK, CHUNK)]], rows.at[buf], sem_g)

      def s_copy(ci, buf):
        return pltpu.make_async_copy(
            rows.at[buf], out_b.at[wcell2.at[ci]], sem_s)

      g_copy(0, 0).start()

      @pl.when(nch > 1)
      def _():
        g_copy(1, 1).start()

      def move(ci, _):
        @pl.when(ci > 1)
        def _():
          s_copy(ci - 2, (ci - 2) % 4).wait()

        @pl.when(ci + 2 < nch)
        def _():
          g_copy(ci + 2, (ci + 2) % 4).start()
        g_copy(ci, ci % 4).wait()
        s_copy(ci, ci % 4).start()
        return 0
      lax.fori_loop(0, nch, move, 0)

      @pl.when(nch > 1)
      def _():
        s_copy(nch - 2, (nch - 2) % 4).wait()
      s_copy(nch - 1, (nch - 1) % 4).wait()

  return body(tokens, idx)


def kernel(tokens, idx, grid_size):
  del grid_size
  return _winner_scatter(tokens, idx.astype(jnp.int32))


# speedup vs baseline: 1.3031x; 1.0149x over previous
import functools

import jax
import jax.numpy as jnp
from jax import lax
from jax.experimental import pallas as pl
from jax.experimental.pallas import tpu as pltpu
from jax.experimental.pallas import tpu_sc as plsc

B, N_TOK, D = 8, 24576, 128
GRID = 49152
NC, NS, L = 2, 16, 16
NW = NC * NS
QPB = NW // B
RANGE = GRID // QPB
CHUNK = 128
NBLK = RANGE // CHUNK
ZGRP = 8
WIN = 2048
NWIN = N_TOK // WIN
VPW = WIN // L
SENT = 0x7FFFFFFF


def _winner_scatter(tokens, idx):
  mesh = plsc.VectorSubcoreMesh(
      core_axis_name="c", subcore_axis_name="s",
      num_cores=NC, num_subcores=NS)

  @functools.partial(
      pl.kernel,
      out_type=jax.ShapeDtypeStruct((B, GRID, D), jnp.float32),
      mesh=mesh,
      compiler_params=pltpu.CompilerParams(needs_layout_passes=False),
      scratch_types=[
          pltpu.VMEM((2, WIN), jnp.int32),
          pltpu.VMEM((RANGE,), jnp.int32),
          pltpu.VMEM((L,), jnp.int32),
          pltpu.VMEM((RANGE + L,), jnp.int32),
          pltpu.VMEM((RANGE + L,), jnp.int32),
          pltpu.VMEM((NBLK, CHUNK), jnp.int32),
          pltpu.VMEM((CHUNK, D), jnp.float32),
          pltpu.VMEM((3, CHUNK, D), jnp.float32),
          pltpu.SemaphoreType.DMA,
          pltpu.SemaphoreType.DMA((NWIN,)),
          pltpu.SemaphoreType.DMA,
          pltpu.SemaphoreType.DMA,
      ],
  )
  def body(tokens_hbm, idx_hbm, out_hbm, idx_win, inv, bounce,
           wtok, wcell, wcell2, zblk, rows, sem_i, sem_z, sem_g, sem_s):
    wid = lax.axis_index("s") * NC + lax.axis_index("c")
    b = wid // QPB
    base = (wid % QPB) * RANGE

    iota = lax.iota(jnp.int32, L)
    zeros16f = jnp.zeros((L,), jnp.float32)
    neg16 = jnp.full((L,), -1, jnp.int32)
    shift_idx = jnp.minimum(iota + 1, L - 1)

    def init_inv(j, _):
      inv[pl.ds(j * L, L)] = neg16
      return 0
    lax.fori_loop(0, RANGE // L, init_inv, 0)

    def init_z(i, _):
      r = i // (D // L)
      c = (i % (D // L)) * L
      zblk[r, pl.ds(c, L)] = zeros16f
      return 0
    lax.fori_loop(0, CHUNK * (D // L), init_z, 0)

    out_b = out_hbm.at[b]
    tok_b = tokens_hbm.at[b]

    def zero_start(g):
      for t in range(ZGRP):
        blk = g * ZGRP + t
        pltpu.async_copy(
            zblk, out_b.at[pl.ds(base + blk * CHUNK, CHUNK)], sem_z.at[g])

    def zero_drain(g):
      for t in range(ZGRP):
        blk = g * ZGRP + t
        pltpu.make_async_copy(
            zblk, out_b.at[pl.ds(base + blk * CHUNK, CHUNK)], sem_z.at[g]).wait()

    def zero_drain_dyn(g):
      def one(t, _):
        blk = g * ZGRP + t
        pltpu.make_async_copy(
            zblk, out_b.at[pl.ds(base + blk * CHUNK, CHUNK)], sem_z.at[g]).wait()
        return 0
      lax.fori_loop(0, ZGRP, one, 0)

    pltpu.async_copy(idx_hbm.at[b, pl.ds(0, WIN)], idx_win.at[0], sem_i)
    for w in range(NWIN):
      if w + 1 < NWIN:
        pltpu.async_copy(idx_hbm.at[b, pl.ds((w + 1) * WIN, WIN)],
                         idx_win.at[(w + 1) % 2], sem_i)
      pltpu.make_async_copy(idx_hbm.at[b, pl.ds(w * WIN, WIN)],
                            idx_win.at[w % 2], sem_i).wait()
      zero_start(w)

      def vreg_body(k, _, w=w):
        v = idx_win[w % 2, pl.ds(k * L, L)]
        local = v - base
        m = (local >= 0) & (local < RANGE)
        p = (w * WIN + k * L) + iota
        key = jnp.where(m, (local << 15) | p, SENT)
        skey, _ = plsc.sort_key_val(key, key)
        bounce[...] = skey
        snext = plsc.load_gather(bounce, [shift_idx])
        keep = ((skey >> 15) != (snext >> 15)) | (iota == L - 1)
        valid = skey != SENT
        plsc.store_scatter(inv, [skey >> 15], skey & 0x7FFF,
                           mask=keep & valid)
        return 0
      lax.fori_loop(0, VPW, vreg_body, 0)
      if w >= 3:
        zero_drain(w - 3)

    def extract(j, cnt):
      v = inv[pl.ds(j * L, L)]
      m = v >= 0
      plsc.store_compressed(wtok.at[pl.ds(cnt, L)], v, mask=m)
      plsc.store_compressed(wcell.at[pl.ds(cnt, L)],
                            base + j * L + iota, mask=m)
      return cnt + jnp.max(plsc.all_reduce_population_count(m))
    cnt = lax.fori_loop(0, RANGE // L, extract, jnp.int32(0))
    zd0 = jnp.int32(NWIN - 3)
    zeros16 = jnp.zeros((L,), jnp.int32)

    @pl.when(cnt > 0)
    def _():
      nch = (cnt + CHUNK - 1) // CHUNK
      pend = nch * CHUNK
      ftok = plsc.load_gather(wtok, [jnp.zeros((L,), jnp.int32)])
      fcell = plsc.load_gather(wcell, [jnp.zeros((L,), jnp.int32)])
      start = (cnt // L) * L

      def pad(t, _):
        off = start + t * L

        @pl.when(off < pend)
        def _():
          m = (off + iota) >= cnt
          wtok[pl.ds(off, L)] = jnp.where(m, ftok, wtok[pl.ds(off, L)])
          wcell[pl.ds(off, L)] = jnp.where(m, fcell, wcell[pl.ds(off, L)])
        return 0
      lax.fori_loop(0, CHUNK // L, pad, 0)

      def repack(i, _):
        r = i // (CHUNK // L)
        c = (i % (CHUNK // L)) * L
        wcell2[r, pl.ds(c, L)] = wcell[pl.ds(i * L, L)]
        return 0
      lax.fori_loop(0, nch * (CHUNK // L), repack, 0)

      def g_copy(ci, buf):
        return pltpu.make_async_copy(
            tok_b.at[wtok.at[pl.ds(ci * CHUNK, CHUNK)]], rows.at[buf], sem_g)

      def s_copy(ci, buf):
        return pltpu.make_async_copy(
            rows.at[buf], out_b.at[wcell2.at[ci]], sem_s)

      g_copy(0, 0).start()

      @pl.when(nch > 1)
      def _():
        g_copy(1, 1).start()

      def move(ci, zd):
        @pl.when(ci > 1)
        def _():
          s_copy(ci - 2, (ci - 2) % 3).wait()

        @pl.when(ci + 2 < nch)
        def _():
          g_copy(ci + 2, (ci + 2) % 3).start()
        g_copy(ci, ci % 3).wait()

        cmax = jnp.max(plsc.load_gather(
            wcell, [jnp.full((L,), (ci + 1) * CHUNK - 1, jnp.int32)]))
        need = jnp.minimum((cmax - base) // (ZGRP * CHUNK) + 1,
                           jnp.int32(NWIN))

        def drain_more(zdc):
          zero_drain_dyn(zdc)
          return zdc + 1
        zd = lax.while_loop(lambda zdc: zdc < need, drain_more, zd)

        s_copy(ci, ci % 3).start()
        return zd
      zd1 = lax.fori_loop(0, nch, move, zd0)

      @pl.when(nch > 1)
      def _():
        s_copy(nch - 2, (nch - 2) % 3).wait()
      s_copy(nch - 1, (nch - 1) % 3).wait()

      def drain_rest(zdc):
        zero_drain_dyn(zdc)
        return zdc + 1
      lax.while_loop(lambda zdc: zdc < NWIN, drain_rest, zd1)

    @pl.when(cnt == 0)
    def _():
      def drain_rest0(zdc):
        zero_drain_dyn(zdc)
        return zdc + 1
      lax.while_loop(lambda zdc: zdc < NWIN, drain_rest0, zd0)

  return body(tokens, idx)


def kernel(tokens, idx, grid_size):
  del grid_size
  return _winner_scatter(tokens, idx.astype(jnp.int32))
